# Initial kernel scaffold; baseline (speedup 1.0000x reference)
import jax
import jax.numpy as jnp
from jax.experimental import pallas as pl


def kernel(idx, emb1, emb2):
    raise NotImplementedError("write your pallas kernel here")


# speedup vs baseline: 1.5341x; 1.5341x over previous
import functools

import jax
import jax.numpy as jnp
from jax import lax
from jax.experimental import pallas as pl
from jax.experimental.pallas import tpu as pltpu
from jax.experimental.pallas import tpu_sc as plsc

NC, NS = 2, 16
NW = NC * NS
B, D = 16384, 128
BPW = B // NW
CHUNK = 128
NCH = BPW // CHUNK

_mesh = plsc.VectorSubcoreMesh(
    core_axis_name="c", subcore_axis_name="s", num_cores=NC, num_subcores=NS
)


@functools.partial(
    pl.kernel,
    out_type=(
        jax.ShapeDtypeStruct((B, D), jnp.float32),
        jax.ShapeDtypeStruct((B, D), jnp.float32),
    ),
    mesh=_mesh,
    scratch_types=[
        pltpu.VMEM((NCH, CHUNK), jnp.int32),
        pltpu.VMEM((CHUNK, D), jnp.float32),
        pltpu.VMEM((CHUNK, D), jnp.float32),
        pltpu.SemaphoreType.DMA,
        pltpu.SemaphoreType.DMA,
    ],
)
def _gather2(idx_hbm, e1_hbm, e2_hbm, o1_hbm, o2_hbm,
             idx_v, buf0, buf1, sem0, sem1):
    wid = lax.axis_index("s") * NC + lax.axis_index("c")
    base = wid * BPW
    pltpu.sync_copy(idx_hbm.at[wid], idx_v)

    bufs = (buf0, buf1)
    sems = (sem0, sem1)
    tabs = (e1_hbm, e2_hbm)
    outs = (o1_hbm, o2_hbm)
    tasks = [(t, c) for t in range(2) for c in range(NCH)]

    def start(i):
        t, c = tasks[i]
        return pltpu.async_copy(tabs[t].at[idx_v.at[c]], bufs[i % 2], sems[i % 2])

    pending = start(0)
    for i in range(len(tasks)):
        nxt = start(i + 1) if i + 1 < len(tasks) else None
        pending.wait()
        t, c = tasks[i]
        pltpu.sync_copy(bufs[i % 2], outs[t].at[pl.ds(base + c * CHUNK, CHUNK)])
        pending = nxt


def kernel(idx, emb1, emb2):
    idx3 = idx.reshape(NW, NCH, CHUNK).astype(jnp.int32)
    return _gather2(idx3, emb1, emb2)


# speedup vs baseline: 1.5675x; 1.0218x over previous
import functools

import jax
import jax.numpy as jnp
from jax import lax
from jax.experimental import pallas as pl
from jax.experimental.pallas import tpu as pltpu
from jax.experimental.pallas import tpu_sc as plsc

NC, NS = 2, 16
NW = NC * NS
B, D = 16384, 128
BPW = B // NW
CHUNK = 128
NCH = BPW // CHUNK

_mesh = plsc.VectorSubcoreMesh(
    core_axis_name="c", subcore_axis_name="s", num_cores=NC, num_subcores=NS
)


@functools.partial(
    pl.kernel,
    out_type=(
        jax.ShapeDtypeStruct((B, D), jnp.float32),
        jax.ShapeDtypeStruct((B, D), jnp.float32),
    ),
    mesh=_mesh,
    scratch_types=[
        pltpu.VMEM((NCH, CHUNK), jnp.int32),
        pltpu.VMEM((CHUNK, D), jnp.float32),
        pltpu.VMEM((CHUNK, D), jnp.float32),
        pltpu.VMEM((CHUNK, D), jnp.float32),
        pltpu.VMEM((CHUNK, D), jnp.float32),
        pltpu.SemaphoreType.DMA,
        pltpu.SemaphoreType.DMA,
        pltpu.SemaphoreType.DMA,
        pltpu.SemaphoreType.DMA,
        pltpu.SemaphoreType.DMA,
        pltpu.SemaphoreType.DMA,
        pltpu.SemaphoreType.DMA,
        pltpu.SemaphoreType.DMA,
    ],
)
def _gather2(idx_hbm, e1_hbm, e2_hbm, o1_hbm, o2_hbm,
             idx_v, b0, b1, b2, b3,
             gs0, gs1, gs2, gs3, ws0, ws1, ws2, ws3):
    wid = lax.axis_index("s") * NC + lax.axis_index("c")
    base = wid * BPW
    pltpu.sync_copy(idx_hbm.at[wid], idx_v)

    NBUF = 4
    bufs = (b0, b1, b2, b3)
    gsems = (gs0, gs1, gs2, gs3)
    wsems = (ws0, ws1, ws2, ws3)
    tabs = (e1_hbm, e2_hbm)
    outs = (o1_hbm, o2_hbm)
    tasks = [(t, c) for t in range(2) for c in range(NCH)]
    NT = len(tasks)

    def start_g(i):
        t, c = tasks[i]
        return pltpu.async_copy(
            tabs[t].at[idx_v.at[c]], bufs[i % NBUF], gsems[i % NBUF])

    def start_w(i):
        t, c = tasks[i]
        return pltpu.async_copy(
            bufs[i % NBUF], outs[t].at[pl.ds(base + c * CHUNK, CHUNK)],
            wsems[i % NBUF])

    gdesc = [None] * NT
    wdesc = [None] * NT
    for i in range(NBUF):
        gdesc[i] = start_g(i)
    for i in range(NT):
        gdesc[i].wait()
        wdesc[i] = start_w(i)
        j = i + NBUF
        if j < NT:
            wdesc[i].wait()
            gdesc[j] = start_g(j)
    for i in range(NT - NBUF, NT):
        wdesc[i].wait()


def kernel(idx, emb1, emb2):
    idx3 = idx.reshape(NW, NCH, CHUNK).astype(jnp.int32)
    return _gather2(idx3, emb1, emb2)


# speedup vs baseline: 1.6088x; 1.0264x over previous
import functools

import jax
import jax.numpy as jnp
from jax import lax
from jax.experimental import pallas as pl
from jax.experimental.pallas import tpu as pltpu
from jax.experimental.pallas import tpu_sc as plsc

NC, NS = 2, 16
NW = NC * NS
B, D = 16384, 128
BPW = B // NW
CHUNK = 128
NCH = BPW // CHUNK

_mesh = plsc.VectorSubcoreMesh(
    core_axis_name="c", subcore_axis_name="s", num_cores=NC, num_subcores=NS
)


@functools.partial(
    pl.kernel,
    out_type=(
        jax.ShapeDtypeStruct((B, D), jnp.float32),
        jax.ShapeDtypeStruct((B, D), jnp.float32),
    ),
    mesh=_mesh,
    scratch_types=(
        [pltpu.VMEM((NCH, CHUNK), jnp.int32)]
        + [pltpu.VMEM((CHUNK, D), jnp.float32) for _ in range(7)]
        + [pltpu.SemaphoreType.DMA for _ in range(14)]
    ),
)
def _gather2(idx_hbm, e1_hbm, e2_hbm, o1_hbm, o2_hbm, idx_v, *sc):
    NBUF = 7
    bufs = sc[:NBUF]
    gsems = sc[NBUF:2 * NBUF]
    wsems = sc[2 * NBUF:]

    wid = lax.axis_index("s") * NC + lax.axis_index("c")
    base = wid * BPW
    pltpu.sync_copy(idx_hbm.at[wid], idx_v)

    tabs = (e1_hbm, e2_hbm)
    outs = (o1_hbm, o2_hbm)
    tasks = [(t, c) for t in range(2) for c in range(NCH)]
    NT = len(tasks)

    def start_g(i):
        t, c = tasks[i]
        return pltpu.async_copy(
            tabs[t].at[idx_v.at[c]], bufs[i % NBUF], gsems[i % NBUF])

    def start_w(i):
        t, c = tasks[i]
        return pltpu.async_copy(
            bufs[i % NBUF], outs[t].at[pl.ds(base + c * CHUNK, CHUNK)],
            wsems[i % NBUF])

    gdesc = [None] * NT
    wdesc = [None] * NT
    for i in range(min(NBUF, NT)):
        gdesc[i] = start_g(i)
    for i in range(NT):
        gdesc[i].wait()
        wdesc[i] = start_w(i)
        j = i + NBUF
        if j < NT:
            wdesc[i].wait()
            gdesc[j] = start_g(j)
    for i in range(max(0, NT - NBUF), NT):
        wdesc[i].wait()


def kernel(idx, emb1, emb2):
    idx3 = idx.reshape(NW, NCH, CHUNK).astype(jnp.int32)
    return _gather2(idx3, emb1, emb2)
